# baseline (speedup 1.0000x reference)
import functools

import jax
import jax.numpy as jnp
from jax import lax
from jax.experimental import pallas as pl
from jax.experimental.pallas import tpu as pltpu
from jax.experimental.pallas import tpu_sc as plsc

ROWS = 131072
COLS = 14
NC = 2
NS = 16
L = 16
NW = NC * NS
TEC_BATCH = ROWS // NW
CHUNK = 1024
NCHUNK = TEC_BATCH // CHUNK
CGROUPS = CHUNK // L


def _tree_softmax_body(x_hbm, out_hbm, xin0, xin1, xout0, xout1,
                       sin0, sin1, sout0, sout1):
    wid = lax.axis_index("s") * NC + lax.axis_index("c")
    b0 = wid * TEC_BATCH

    xins, xouts = (xin0, xin1), (xout0, xout1)
    sins, souts = (sin0, sin1), (sout0, sout1)

    def src(k):
        return x_hbm.at[:, pl.ds(b0 + k * CHUNK, CHUNK)]

    def dst(k):
        return out_hbm.at[:, pl.ds(b0 + k * CHUNK, CHUNK)]

    cp_in = [None] * NCHUNK
    cp_out = [None] * NCHUNK
    cp_in[0] = pltpu.async_copy(src(0), xins[0], sins[0])
    for k in range(NCHUNK):
        if k + 1 < NCHUNK:
            cp_in[k + 1] = pltpu.async_copy(
                src(k + 1), xins[(k + 1) % 2], sins[(k + 1) % 2])
        cp_in[k].wait()
        if k >= 2:
            cp_out[k - 2].wait()
        xin = xins[k % 2]
        xout = xouts[k % 2]

        @plsc.parallel_loop(0, CGROUPS, step=1, unroll=4)
        def group(g):
            sl = pl.ds(g * L, L)
            c = [xin[j, sl] for j in range(COLS)]
            s = [None] * COLS
            for kk in range(COLS // 2):
                e = jnp.exp(c[2 * kk + 1] - c[2 * kk])
                sa = 1.0 / (1.0 + e)
                s[2 * kk] = sa
                s[2 * kk + 1] = 1.0 - sa
            o = [None] * COLS
            o[0], o[1] = s[0], s[1]
            for j in range(2, COLS):
                o[j] = s[j] * o[j // 2 - 1]
            for j in range(COLS):
                xout[j, sl] = o[j]

        cp_out[k] = pltpu.async_copy(xout, dst(k), souts[k % 2])
    cp_out[NCHUNK - 2].wait()
    cp_out[NCHUNK - 1].wait()


@jax.jit
def kernel(input):
    mesh = plsc.VectorSubcoreMesh(core_axis_name="c", subcore_axis_name="s")
    run = pl.kernel(
        _tree_softmax_body,
        out_type=jax.ShapeDtypeStruct((COLS, ROWS), jnp.float32),
        mesh=mesh,
        scratch_types=[
            pltpu.VMEM((COLS, CHUNK), jnp.float32),
            pltpu.VMEM((COLS, CHUNK), jnp.float32),
            pltpu.VMEM((COLS, CHUNK), jnp.float32),
            pltpu.VMEM((COLS, CHUNK), jnp.float32),
            pltpu.SemaphoreType.DMA,
            pltpu.SemaphoreType.DMA,
            pltpu.SemaphoreType.DMA,
            pltpu.SemaphoreType.DMA,
        ],
        compiler_params=pltpu.CompilerParams(
            needs_layout_passes=False,
            use_tc_tiling_on_sc=False,
            disable_bounds_checks=True,
            disable_semaphore_checks=True,
            skip_device_barrier=True,
        ),
    )
    return run(jnp.transpose(input)).T


# speedup vs baseline: 1.0149x; 1.0149x over previous
import jax
import jax.numpy as jnp
from jax import lax
from jax.experimental import pallas as pl
from jax.experimental.pallas import tpu as pltpu
from jax.experimental.pallas import tpu_sc as plsc

ROWS = 131072
COLS = 14
NC = 2
NS = 16
L = 16
NW = NC * NS
TEC_BATCH = ROWS // NW
CHUNK = 2048
NCHUNK = TEC_BATCH // CHUNK
CGROUPS = CHUNK // L


def _tree_softmax_body(x_hbm, out_hbm, xin0, xin1, xout0, xout1,
                       sin0, sin1, sout0, sout1):
    wid = lax.axis_index("s") * NC + lax.axis_index("c")
    b0 = wid * TEC_BATCH

    xins, xouts = (xin0, xin1), (xout0, xout1)
    sins, souts = (sin0, sin1), (sout0, sout1)

    def src(k):
        return x_hbm.at[:, pl.ds(b0 + k * CHUNK, CHUNK)]

    def dst(k):
        return out_hbm.at[:, pl.ds(b0 + k * CHUNK, CHUNK)]

    cp_in = [None] * NCHUNK
    cp_out = [None] * NCHUNK
    cp_in[0] = pltpu.async_copy(src(0), xins[0], sins[0])
    for k in range(NCHUNK):
        if k + 1 < NCHUNK:
            cp_in[k + 1] = pltpu.async_copy(
                src(k + 1), xins[(k + 1) % 2], sins[(k + 1) % 2])
        cp_in[k].wait()
        if k >= 2:
            cp_out[k - 2].wait()
        xin = xins[k % 2]
        xout = xouts[k % 2]

        @plsc.parallel_loop(0, CGROUPS, step=1, unroll=4)
        def group(g):
            sl = pl.ds(g * L, L)
            c = [xin[j, sl] for j in range(COLS)]
            s = [None] * COLS
            for kk in range(COLS // 2):
                e = jnp.exp(c[2 * kk + 1] - c[2 * kk])
                sa = 1.0 / (1.0 + e)
                s[2 * kk] = sa
                s[2 * kk + 1] = 1.0 - sa
            o = [None] * COLS
            o[0], o[1] = s[0], s[1]
            for j in range(2, COLS):
                o[j] = s[j] * o[j // 2 - 1]
            for j in range(COLS):
                xout[j, sl] = o[j]

        cp_out[k] = pltpu.async_copy(xout, dst(k), souts[k % 2])
    cp_out[NCHUNK - 2].wait()
    cp_out[NCHUNK - 1].wait()


@jax.jit
def kernel(input):
    mesh = plsc.VectorSubcoreMesh(core_axis_name="c", subcore_axis_name="s")
    run = pl.kernel(
        _tree_softmax_body,
        out_type=jax.ShapeDtypeStruct((COLS, ROWS), jnp.float32),
        mesh=mesh,
        scratch_types=[
            pltpu.VMEM((COLS, CHUNK), jnp.float32),
            pltpu.VMEM((COLS, CHUNK), jnp.float32),
            pltpu.VMEM((COLS, CHUNK), jnp.float32),
            pltpu.VMEM((COLS, CHUNK), jnp.float32),
            pltpu.SemaphoreType.DMA,
            pltpu.SemaphoreType.DMA,
            pltpu.SemaphoreType.DMA,
            pltpu.SemaphoreType.DMA,
        ],
        compiler_params=pltpu.CompilerParams(
            needs_layout_passes=False,
            use_tc_tiling_on_sc=False,
            disable_bounds_checks=True,
            disable_semaphore_checks=True,
            skip_device_barrier=True,
        ),
    )
    return run(jnp.transpose(input)).T
